# baseline (speedup 1.0000x reference)
import functools

import jax
import jax.numpy as jnp
from jax import lax
from jax.experimental import pallas as pl
from jax.experimental.pallas import tpu as pltpu
from jax.experimental.pallas import tpu_sc as plsc

B = 4
S = 4096
D = 2048
E = 64
K = 8

NW = 32
ROWS_PW = (B * S) // NW
CH = 16
NCH = ROWS_PW // CH


def _sc_mean_body(vid, out, buf0, buf1, acc, sem0, sem1):
    wid = lax.axis_index("c") * 16 + lax.axis_index("s")
    b = wid // (NW // B)
    sbase = (wid % (NW // B)) * ROWS_PW

    def zbody(i, c):
        acc[pl.ds(i * 16, 16)] = jnp.zeros((16,), jnp.float32)
        return c
    lax.fori_loop(0, D // 16, zbody, 0)

    def accum(buf):
        def ibody(i, c):
            off = i * 16
            s = buf[0, pl.ds(off, 16)]
            for r in range(1, CH):
                s = s + buf[r, pl.ds(off, 16)]
            acc[pl.ds(off, 16)] += s
            return c
        lax.fori_loop(0, D // 16, ibody, 0)

    def start(buf, sem, c):
        pltpu.make_async_copy(
            vid.at[b, pl.ds(sbase + c * CH, CH)], buf, sem).start()

    def wait(buf, sem):
        pltpu.make_async_copy(vid.at[b, pl.ds(sbase, CH)], buf, sem).wait()

    start(buf0, sem0, 0)

    def pair(p, c):
        start(buf1, sem1, 2 * p + 1)
        wait(buf0, sem0)
        accum(buf0)

        @pl.when(2 * p + 2 < NCH)
        def _():
            start(buf0, sem0, 2 * p + 2)
        wait(buf1, sem1)
        accum(buf1)
        return c
    lax.fori_loop(0, NCH // 2, pair, 0)

    pltpu.sync_copy(acc, out.at[wid])


_sc_mean = pl.kernel(
    _sc_mean_body,
    out_type=jax.ShapeDtypeStruct((NW, D), jnp.float32),
    mesh=plsc.VectorSubcoreMesh(core_axis_name="c", subcore_axis_name="s"),
    scratch_types=[
        pltpu.VMEM((CH, D), jnp.float32),
        pltpu.VMEM((CH, D), jnp.float32),
        pltpu.VMEM((D,), jnp.float32),
        pltpu.SemaphoreType.DMA,
        pltpu.SemaphoreType.DMA,
    ],
)


def _head_body(hpart_ref, text_ref, w1_ref, w2_ref, b_ref,
               topi_ref, topv_ref, probs_ref, ent_ref, aux_ref):
    h = jnp.sum(hpart_ref[...].reshape(B, NW // B, D), axis=1) * (1.0 / S)
    logits = (jnp.dot(h, w1_ref[...], preferred_element_type=jnp.float32)
              + jnp.dot(text_ref[...], w2_ref[...],
                        preferred_element_type=jnp.float32)
              + b_ref[...])
    m = jnp.max(logits, axis=-1, keepdims=True)
    ex = jnp.exp(logits - m)
    probs = ex / jnp.sum(ex, axis=-1, keepdims=True)
    probs_ref[...] = probs

    ent = -jnp.sum(probs * jnp.log(probs + 1e-8)) * (1.0 / B)
    ent_ref[...] = ent.reshape(1, 1)
    mu = jnp.mean(probs, axis=0, keepdims=True)
    aux_ref[...] = jnp.mean((probs - mu) ** 2).reshape(1, 1)

    idxs = lax.broadcasted_iota(jnp.int32, (B, E), 1)
    work = probs
    vals = []
    args = []
    for _ in range(K):
        cur = jnp.max(work, axis=-1, keepdims=True)
        is_max = work == cur
        arg = jnp.min(jnp.where(is_max, idxs, E), axis=-1,
                      keepdims=True)
        vals.append(cur)
        args.append(arg)
        work = jnp.where(idxs == arg, -jnp.inf, work)
    topv = jnp.concatenate(vals, axis=1)
    topi = jnp.concatenate(args, axis=1)
    topi_ref[...] = topi
    topv_ref[...] = topv / (jnp.sum(topv, axis=-1, keepdims=True) + 1e-8)


def _head(hpart, text_state, w1, w2, b2):
    return pl.pallas_call(
        _head_body,
        out_shape=[
            jax.ShapeDtypeStruct((B, K), jnp.int32),
            jax.ShapeDtypeStruct((B, K), jnp.float32),
            jax.ShapeDtypeStruct((B, E), jnp.float32),
            jax.ShapeDtypeStruct((1, 1), jnp.float32),
            jax.ShapeDtypeStruct((1, 1), jnp.float32),
        ],
    )(hpart, text_state, w1, w2, b2)


@functools.partial(jax.jit, static_argnames=())
def kernel(video_tokens, text_state, W, b):
    w1 = W[:D]
    w2 = W[D:]
    b2 = b.reshape(1, E)
    hpart = _sc_mean(video_tokens)
    topi, topv, probs, ent, aux = _head(hpart, text_state, w1, w2, b2)
    return (topi, topv, probs, ent.reshape(()), aux.reshape(()))


# speedup vs baseline: 1.8168x; 1.8168x over previous
import functools

import jax
import jax.numpy as jnp
from jax import lax
from jax.experimental import pallas as pl
from jax.experimental.pallas import tpu as pltpu
from jax.experimental.pallas import tpu_sc as plsc

B = 4
S = 4096
D = 2048
E = 64
K = 8

B_TC = 3
SBLK = 256
NBLK = S // SBLK

NW = 32
SC_B0 = B_TC
ROWS_PW = ((B - B_TC) * S) // NW
CH = 16
NCH = ROWS_PW // CH


def _sc_mean_body(vid, out, buf0, buf1, acc, sem0, sem1):
    wid = lax.axis_index("c") * 16 + lax.axis_index("s")
    nw_per_b = NW // (B - B_TC)
    b = SC_B0 + wid // nw_per_b
    sbase = (wid % nw_per_b) * ROWS_PW

    def zbody(i, c):
        acc[pl.ds(i * 16, 16)] = jnp.zeros((16,), jnp.float32)
        return c
    lax.fori_loop(0, D // 16, zbody, 0)

    def accum(buf):
        def ibody(i, c):
            off = i * 16
            s = buf[0, pl.ds(off, 16)]
            for r in range(1, CH):
                s = s + buf[r, pl.ds(off, 16)]
            acc[pl.ds(off, 16)] += s
            return c
        lax.fori_loop(0, D // 16, ibody, 0)

    def start(buf, sem, c):
        pltpu.make_async_copy(
            vid.at[b, pl.ds(sbase + c * CH, CH)], buf, sem).start()

    def wait(buf, sem):
        pltpu.make_async_copy(vid.at[b, pl.ds(sbase, CH)], buf, sem).wait()

    start(buf0, sem0, 0)

    def pair(p, c):
        start(buf1, sem1, 2 * p + 1)
        wait(buf0, sem0)
        accum(buf0)

        @pl.when(2 * p + 2 < NCH)
        def _():
            start(buf0, sem0, 2 * p + 2)
        wait(buf1, sem1)
        accum(buf1)
        return c
    lax.fori_loop(0, NCH // 2, pair, 0)

    pltpu.sync_copy(acc, out.at[wid])


_sc_mean = pl.kernel(
    _sc_mean_body,
    out_type=jax.ShapeDtypeStruct((NW, D), jnp.float32),
    mesh=plsc.VectorSubcoreMesh(core_axis_name="c", subcore_axis_name="s"),
    scratch_types=[
        pltpu.VMEM((CH, D), jnp.float32),
        pltpu.VMEM((CH, D), jnp.float32),
        pltpu.VMEM((D,), jnp.float32),
        pltpu.SemaphoreType.DMA,
        pltpu.SemaphoreType.DMA,
    ],
)


def _tc_mean_body(vt_ref, hsum_ref, acc_ref):
    i = pl.program_id(0)

    @pl.when(i == 0)
    def _init():
        acc_ref[...] = jnp.zeros_like(acc_ref)

    acc_ref[...] += jnp.sum(vt_ref[...], axis=1)

    @pl.when(i == NBLK - 1)
    def _emit():
        hsum_ref[...] = acc_ref[...]


def _tc_mean(video_tokens):
    return pl.pallas_call(
        _tc_mean_body,
        grid=(NBLK,),
        in_specs=[pl.BlockSpec((B_TC, SBLK, D), lambda i: (0, i, 0))],
        out_specs=pl.BlockSpec((B_TC, D), lambda i: (0, 0)),
        out_shape=jax.ShapeDtypeStruct((B_TC, D), jnp.float32),
        scratch_shapes=[pltpu.VMEM((B_TC, D), jnp.float32)],
    )(video_tokens)


def _head_body(hsum_ref, hpart_ref, text_ref, w1_ref, w2_ref, b_ref,
               topi_ref, topv_ref, probs_ref, ent_ref, aux_ref):
    h_sc = jnp.sum(hpart_ref[...].reshape(B - B_TC, NW // (B - B_TC), D),
                   axis=1)
    h = jnp.concatenate([hsum_ref[...], h_sc], axis=0) * (1.0 / S)
    logits = (jnp.dot(h, w1_ref[...], preferred_element_type=jnp.float32)
              + jnp.dot(text_ref[...], w2_ref[...],
                        preferred_element_type=jnp.float32)
              + b_ref[...])
    m = jnp.max(logits, axis=-1, keepdims=True)
    ex = jnp.exp(logits - m)
    probs = ex / jnp.sum(ex, axis=-1, keepdims=True)
    probs_ref[...] = probs

    ent = -jnp.sum(probs * jnp.log(probs + 1e-8)) * (1.0 / B)
    ent_ref[...] = ent.reshape(1, 1)
    mu = jnp.mean(probs, axis=0, keepdims=True)
    aux_ref[...] = jnp.mean((probs - mu) ** 2).reshape(1, 1)

    idxs = lax.broadcasted_iota(jnp.int32, (B, E), 1)
    work = probs
    vals = []
    args = []
    for _ in range(K):
        cur = jnp.max(work, axis=-1, keepdims=True)
        is_max = work == cur
        arg = jnp.min(jnp.where(is_max, idxs, E), axis=-1,
                      keepdims=True)
        vals.append(cur)
        args.append(arg)
        work = jnp.where(idxs == arg, -jnp.inf, work)
    topv = jnp.concatenate(vals, axis=1)
    topi = jnp.concatenate(args, axis=1)
    topi_ref[...] = topi
    topv_ref[...] = topv / (jnp.sum(topv, axis=-1, keepdims=True) + 1e-8)


def _head(hsum, hpart, text_state, w1, w2, b2):
    return pl.pallas_call(
        _head_body,
        out_shape=[
            jax.ShapeDtypeStruct((B, K), jnp.int32),
            jax.ShapeDtypeStruct((B, K), jnp.float32),
            jax.ShapeDtypeStruct((B, E), jnp.float32),
            jax.ShapeDtypeStruct((1, 1), jnp.float32),
            jax.ShapeDtypeStruct((1, 1), jnp.float32),
        ],
    )(hsum, hpart, text_state, w1, w2, b2)


@functools.partial(jax.jit, static_argnames=())
def kernel(video_tokens, text_state, W, b):
    w1 = W[:D]
    w2 = W[D:]
    b2 = b.reshape(1, E)
    hpart = _sc_mean(video_tokens)
    hsum = _tc_mean(video_tokens)
    topi, topv, probs, ent, aux = _head(hsum, hpart, text_state, w1, w2, b2)
    return (topi, topv, probs, ent.reshape(()), aux.reshape(()))


# speedup vs baseline: 2.4313x; 1.3382x over previous
import functools

import jax
import jax.numpy as jnp
from jax.experimental import pallas as pl
from jax.experimental.pallas import tpu as pltpu

B = 4
S = 4096
D = 2048
E = 64
K = 8
SBLK = 1024
NBLK = S // SBLK


def _router_body(vt_ref, text_ref, w1_ref, w2_ref, b_ref,
                 topi_ref, topv_ref, probs_ref, ent_ref, aux_ref,
                 acc_ref, hrows_ref):
    bidx = pl.program_id(0)
    sidx = pl.program_id(1)

    @pl.when(sidx == 0)
    def _init():
        acc_ref[...] = jnp.zeros_like(acc_ref)

    acc_ref[...] += jnp.sum(vt_ref[0].reshape(SBLK // 8, 8, D), axis=0)

    for k in range(B):
        @pl.when((bidx == k) & (sidx == NBLK - 1))
        def _stash():
            hrows_ref[8 * k:8 * k + 8, :] = acc_ref[...]

    @pl.when((bidx == B - 1) & (sidx == NBLK - 1))
    def _finish():
        h = jnp.sum(hrows_ref[...].reshape(B, 8, D), axis=1) * (1.0 / S)
        logits = (jnp.dot(h, w1_ref[...], preferred_element_type=jnp.float32)
                  + jnp.dot(text_ref[...], w2_ref[...],
                            preferred_element_type=jnp.float32)
                  + b_ref[...])
        m = jnp.max(logits, axis=-1, keepdims=True)
        ex = jnp.exp(logits - m)
        probs = ex / jnp.sum(ex, axis=-1, keepdims=True)
        probs_ref[...] = probs

        ent = -jnp.sum(probs * jnp.log(probs + 1e-8)) * (1.0 / B)
        ent_ref[...] = ent.reshape(1, 1)
        mu = jnp.mean(probs, axis=0, keepdims=True)
        aux_ref[...] = jnp.mean((probs - mu) ** 2).reshape(1, 1)

        idxs = jax.lax.broadcasted_iota(jnp.int32, (B, E), 1)
        work = probs
        vals = []
        args = []
        for _ in range(K):
            cur = jnp.max(work, axis=-1, keepdims=True)
            is_max = work == cur
            arg = jnp.min(jnp.where(is_max, idxs, E), axis=-1,
                          keepdims=True)
            vals.append(cur)
            args.append(arg)
            work = jnp.where(idxs == arg, -jnp.inf, work)
        topv = jnp.concatenate(vals, axis=1)
        topi = jnp.concatenate(args, axis=1)
        topi_ref[...] = topi
        topv_ref[...] = topv / (jnp.sum(topv, axis=-1, keepdims=True) + 1e-8)


@functools.partial(jax.jit, static_argnames=())
def kernel(video_tokens, text_state, W, b):
    w1 = W[:D]
    w2 = W[D:]
    b2 = b.reshape(1, E)
    grid = (B, NBLK)
    topi, topv, probs, ent, aux = pl.pallas_call(
        _router_body,
        grid=grid,
        in_specs=[
            pl.BlockSpec((1, SBLK, D), lambda bi, si: (bi, si, 0)),
            pl.BlockSpec((B, D), lambda bi, si: (0, 0)),
            pl.BlockSpec((D, E), lambda bi, si: (0, 0)),
            pl.BlockSpec((D, E), lambda bi, si: (0, 0)),
            pl.BlockSpec((1, E), lambda bi, si: (0, 0)),
        ],
        out_specs=[
            pl.BlockSpec((B, K), lambda bi, si: (0, 0)),
            pl.BlockSpec((B, K), lambda bi, si: (0, 0)),
            pl.BlockSpec((B, E), lambda bi, si: (0, 0)),
            pl.BlockSpec((1, 1), lambda bi, si: (0, 0)),
            pl.BlockSpec((1, 1), lambda bi, si: (0, 0)),
        ],
        out_shape=[
            jax.ShapeDtypeStruct((B, K), jnp.int32),
            jax.ShapeDtypeStruct((B, K), jnp.float32),
            jax.ShapeDtypeStruct((B, E), jnp.float32),
            jax.ShapeDtypeStruct((1, 1), jnp.float32),
            jax.ShapeDtypeStruct((1, 1), jnp.float32),
        ],
        scratch_shapes=[pltpu.VMEM((8, D), jnp.float32),
                        pltpu.VMEM((8 * B, D), jnp.float32)],
    )(video_tokens, text_state, w1, w2, b2)
    return (topi, topv, probs, ent.reshape(()), aux.reshape(()))


# speedup vs baseline: 2.4800x; 1.0201x over previous
import functools

import jax
import jax.numpy as jnp
from jax.experimental import pallas as pl
from jax.experimental.pallas import tpu as pltpu

B = 4
S = 4096
D = 2048
E = 64
K = 8
SBLK = 2048
NBLK = S // SBLK


def _router_body(vt_ref, text_ref, w1_ref, w2_ref, b_ref,
                 topi_ref, topv_ref, probs_ref, ent_ref, aux_ref,
                 acc_ref, hrows_ref):
    bidx = pl.program_id(0)
    sidx = pl.program_id(1)

    @pl.when(sidx == 0)
    def _init():
        acc_ref[...] = jnp.zeros_like(acc_ref)

    acc_ref[...] += jnp.sum(vt_ref[0].reshape(SBLK // 8, 8, D), axis=0)

    for k in range(B):
        @pl.when((bidx == k) & (sidx == NBLK - 1))
        def _stash():
            hrows_ref[8 * k:8 * k + 8, :] = acc_ref[...]

    @pl.when((bidx == B - 1) & (sidx == NBLK - 1))
    def _finish():
        h = jnp.sum(hrows_ref[...].reshape(B, 8, D), axis=1) * (1.0 / S)
        logits = (jnp.dot(h, w1_ref[...], preferred_element_type=jnp.float32)
                  + jnp.dot(text_ref[...], w2_ref[...],
                            preferred_element_type=jnp.float32)
                  + b_ref[...])
        m = jnp.max(logits, axis=-1, keepdims=True)
        ex = jnp.exp(logits - m)
        probs = ex / jnp.sum(ex, axis=-1, keepdims=True)
        probs_ref[...] = probs

        ent = -jnp.sum(probs * jnp.log(probs + 1e-8)) * (1.0 / B)
        ent_ref[...] = ent.reshape(1, 1)
        mu = jnp.mean(probs, axis=0, keepdims=True)
        aux_ref[...] = jnp.mean((probs - mu) ** 2).reshape(1, 1)

        idxs = jax.lax.broadcasted_iota(jnp.int32, (B, E), 1)
        work = probs
        vals = []
        args = []
        for _ in range(K):
            cur = jnp.max(work, axis=-1, keepdims=True)
            is_max = work == cur
            arg = jnp.min(jnp.where(is_max, idxs, E), axis=-1,
                          keepdims=True)
            vals.append(cur)
            args.append(arg)
            work = jnp.where(idxs == arg, -jnp.inf, work)
        topv = jnp.concatenate(vals, axis=1)
        topi = jnp.concatenate(args, axis=1)
        topi_ref[...] = topi
        topv_ref[...] = topv / (jnp.sum(topv, axis=-1, keepdims=True) + 1e-8)


@functools.partial(jax.jit, static_argnames=())
def kernel(video_tokens, text_state, W, b):
    w1 = W[:D]
    w2 = W[D:]
    b2 = b.reshape(1, E)
    grid = (B, NBLK)
    topi, topv, probs, ent, aux = pl.pallas_call(
        _router_body,
        grid=grid,
        in_specs=[
            pl.BlockSpec((1, SBLK, D), lambda bi, si: (bi, si, 0)),
            pl.BlockSpec((B, D), lambda bi, si: (0, 0)),
            pl.BlockSpec((D, E), lambda bi, si: (0, 0)),
            pl.BlockSpec((D, E), lambda bi, si: (0, 0)),
            pl.BlockSpec((1, E), lambda bi, si: (0, 0)),
        ],
        out_specs=[
            pl.BlockSpec((B, K), lambda bi, si: (0, 0)),
            pl.BlockSpec((B, K), lambda bi, si: (0, 0)),
            pl.BlockSpec((B, E), lambda bi, si: (0, 0)),
            pl.BlockSpec((1, 1), lambda bi, si: (0, 0)),
            pl.BlockSpec((1, 1), lambda bi, si: (0, 0)),
        ],
        out_shape=[
            jax.ShapeDtypeStruct((B, K), jnp.int32),
            jax.ShapeDtypeStruct((B, K), jnp.float32),
            jax.ShapeDtypeStruct((B, E), jnp.float32),
            jax.ShapeDtypeStruct((1, 1), jnp.float32),
            jax.ShapeDtypeStruct((1, 1), jnp.float32),
        ],
        scratch_shapes=[pltpu.VMEM((8, D), jnp.float32),
                        pltpu.VMEM((8 * B, D), jnp.float32)],
    )(video_tokens, text_state, w1, w2, b2)
    return (topi, topv, probs, ent.reshape(()), aux.reshape(()))
